# baseline (speedup 1.0000x reference)
import functools

import jax
import jax.numpy as jnp
from jax import lax
from jax.experimental import pallas as pl
from jax.experimental.pallas import tpu as pltpu
from jax.experimental.pallas import tpu_sc as plsc

NN = 10000
NE = 320000
D = 128
NG = 64

NC, NS = 2, 16
NW = NC * NS
NNP = 10240
RPT = NNP // NS
NR = NNP // D

EPT = 10240
NEP = NW * EPT
CH = 80
G = 4
GE = G * CH
NGRP = EPT // GE
EPT0 = NE // NW
DGE = 400
DGRP = EPT0 // DGE

BLK = 1024
NBLK = NNP // BLK
SUB = BLK // D

_f32 = jnp.float32
_i32 = jnp.int32

_mesh = plsc.VectorSubcoreMesh(core_axis_name="c", subcore_axis_name="s")


@functools.partial(
    pl.kernel,
    out_type=jax.ShapeDtypeStruct((NC, NNP, D), _f32),
    mesh=_mesh,
    scratch_types=[
        pltpu.VMEM((CH,), _i32),
        pltpu.VMEM((CH,), _i32),
        pltpu.VMEM((CH,), _i32),
        pltpu.VMEM((CH,), _i32),
        pltpu.VMEM((CH, D), _f32),
        pltpu.VMEM_SHARED((NNP, D), _f32),
        pltpu.SemaphoreType.DMA,
        pltpu.SemaphoreType.DMA,
    ],
)
def _sc_degree(dst_hbm, zeros_hbm, ones_hbm, out_hbm,
               d0, d1, d2, d3, ones_v, accum, isem, ssem):
    dst_bufs = [d0, d1, d2, d3]
    c = lax.axis_index("c")
    s = lax.axis_index("s")
    wid = c * NS + s
    pltpu.sync_copy(ones_hbm, ones_v)
    pltpu.sync_copy(zeros_hbm, accum.at[pl.ds(s * RPT, RPT)])
    plsc.subcore_barrier()

    def group(g, carry):
        base = wid * EPT + g * GE
        ic = [pltpu.async_copy(dst_hbm.at[pl.ds(base + b * CH, CH)],
                               dst_bufs[b], isem) for b in range(G)]
        for c_ in ic:
            c_.wait()
        sc = [pltpu.async_copy(ones_v, accum.at[dst_bufs[b]], ssem, add=True)
              for b in range(G)]
        for c_ in sc:
            c_.wait()
        return carry

    lax.fori_loop(0, NGRP, group, 0)
    plsc.subcore_barrier()
    pltpu.sync_copy(accum.at[pl.ds(s * RPT, RPT)],
                    out_hbm.at[c, pl.ds(s * RPT, RPT)])


@functools.partial(
    pl.kernel,
    out_type=jax.ShapeDtypeStruct((NC, NNP, D), _f32),
    mesh=_mesh,
    scratch_types=[
        pltpu.VMEM((CH,), _i32),
        pltpu.VMEM((CH,), _i32),
        pltpu.VMEM((CH,), _i32),
        pltpu.VMEM((CH,), _i32),
        pltpu.VMEM((CH,), _i32),
        pltpu.VMEM((CH,), _i32),
        pltpu.VMEM((CH,), _i32),
        pltpu.VMEM((CH,), _i32),
        pltpu.VMEM((CH, D), _f32),
        pltpu.VMEM((CH, D), _f32),
        pltpu.VMEM((CH, D), _f32),
        pltpu.VMEM((CH, D), _f32),
        pltpu.VMEM_SHARED((NNP, D), _f32),
        pltpu.SemaphoreType.DMA,
        pltpu.SemaphoreType.DMA,
        pltpu.SemaphoreType.DMA,
    ],
)
def _sc_agg(src_hbm, dst_hbm, u_hbm, zeros_hbm, out_hbm,
            s0, s1, s2, s3, d0, d1, d2, d3, r0, r1, r2, r3,
            accum, isem, gsem, ssem):
    src_bufs = [s0, s1, s2, s3]
    dst_bufs = [d0, d1, d2, d3]
    rows_bufs = [r0, r1, r2, r3]
    c = lax.axis_index("c")
    s = lax.axis_index("s")
    wid = c * NS + s
    pltpu.sync_copy(zeros_hbm, accum.at[pl.ds(s * RPT, RPT)])
    plsc.subcore_barrier()

    def group(g, carry):
        base = wid * EPT + g * GE
        ic = [pltpu.async_copy(src_hbm.at[pl.ds(base + b * CH, CH)],
                               src_bufs[b], isem) for b in range(G)]
        ic += [pltpu.async_copy(dst_hbm.at[pl.ds(base + b * CH, CH)],
                                dst_bufs[b], isem) for b in range(G)]
        for c_ in ic:
            c_.wait()
        gc = [pltpu.async_copy(u_hbm.at[src_bufs[b]], rows_bufs[b], gsem)
              for b in range(G)]
        for c_ in gc:
            c_.wait()
        sc = [pltpu.async_copy(rows_bufs[b], accum.at[dst_bufs[b]], ssem,
                               add=True) for b in range(G)]
        for c_ in sc:
            c_.wait()
        return carry

    lax.fori_loop(0, NGRP, group, 0)
    plsc.subcore_barrier()
    pltpu.sync_copy(accum.at[pl.ds(s * RPT, RPT)],
                    out_hbm.at[c, pl.ds(s * RPT, RPT)])


def _ident128():
    a = lax.broadcasted_iota(_i32, (D, D), 0)
    b = lax.broadcasted_iota(_i32, (D, D), 1)
    return (a == b).astype(_f32)


def _tc1_body(degc_ref, x_ref, w1_ref, u1_ref, dinvb_ref):
    deg = degc_ref[0, :, 0:1] + degc_ref[1, :, 0:1] + 1.0
    dinv = lax.rsqrt(deg)
    h = jnp.dot(x_ref[...], w1_ref[...], preferred_element_type=_f32)
    u1_ref[...] = h * dinv
    dinvb_ref[...] = jnp.broadcast_to(dinv, (BLK, D))


_tc1 = pl.pallas_call(
    _tc1_body,
    grid=(NBLK,),
    in_specs=[
        pl.BlockSpec((NC, BLK, D), lambda i: (0, i, 0)),
        pl.BlockSpec((BLK, D), lambda i: (i, 0)),
        pl.BlockSpec((D, D), lambda i: (0, 0)),
    ],
    out_specs=[
        pl.BlockSpec((BLK, D), lambda i: (i, 0)),
        pl.BlockSpec((BLK, D), lambda i: (i, 0)),
    ],
    out_shape=[
        jax.ShapeDtypeStruct((NNP, D), _f32),
        jax.ShapeDtypeStruct((NNP, D), _f32),
    ],
)


def _tc2_body(aggp_ref, u1_ref, dinvb_ref, w2_ref, b1_ref, u2_ref):
    dinvb = dinvb_ref[...]
    t = aggp_ref[0] + aggp_ref[1] + u1_ref[...]
    out1 = jnp.maximum(t * dinvb + b1_ref[...], 0.0)
    h2 = jnp.dot(out1, w2_ref[...], preferred_element_type=_f32)
    u2_ref[...] = h2 * dinvb


_tc2 = pl.pallas_call(
    _tc2_body,
    grid=(NBLK,),
    in_specs=[
        pl.BlockSpec((NC, BLK, D), lambda i: (0, i, 0)),
        pl.BlockSpec((BLK, D), lambda i: (i, 0)),
        pl.BlockSpec((BLK, D), lambda i: (i, 0)),
        pl.BlockSpec((D, D), lambda i: (0, 0)),
        pl.BlockSpec((1, D), lambda i: (0, 0)),
    ],
    out_specs=pl.BlockSpec((BLK, D), lambda i: (i, 0)),
    out_shape=jax.ShapeDtypeStruct((NNP, D), _f32),
)


def _tc3_body(aggp_ref, u2_ref, dinvb_ref, b2_ref, batch_ref, out_ref,
              acc_ref, cnt_ref):
    i = pl.program_id(0)

    @pl.when(i == 0)
    def _():
        acc_ref[...] = jnp.zeros_like(acc_ref)
        cnt_ref[...] = jnp.zeros_like(cnt_ref)

    t = (aggp_ref[0] + aggp_ref[1] + u2_ref[...]) * dinvb_ref[...]
    b = batch_ref[0]
    gids = lax.broadcasted_iota(_i32, (BLK, NG), 1)
    onehot = (b == gids).astype(_f32)
    acc_ref[...] += lax.dot_general(
        onehot, t, (((0,), (0,)), ((), ())), preferred_element_type=_f32)
    cnt_ref[...] += lax.dot_general(
        onehot, jnp.ones((BLK, D), _f32), (((0,), (0,)), ((), ())),
        preferred_element_type=_f32)

    @pl.when(i == pl.num_programs(0) - 1)
    def _():
        out_ref[...] = acc_ref[...] / jnp.maximum(cnt_ref[...], 1.0) + b2_ref[...]


_tc3 = pl.pallas_call(
    _tc3_body,
    grid=(NBLK,),
    in_specs=[
        pl.BlockSpec((NC, BLK, D), lambda i: (0, i, 0)),
        pl.BlockSpec((BLK, D), lambda i: (i, 0)),
        pl.BlockSpec((BLK, D), lambda i: (i, 0)),
        pl.BlockSpec((1, D), lambda i: (0, 0)),
        pl.BlockSpec((1, BLK, 1), lambda i: (i, 0, 0)),
    ],
    out_specs=pl.BlockSpec((NG, D), lambda i: (0, 0)),
    out_shape=jax.ShapeDtypeStruct((NG, D), _f32),
    scratch_shapes=[
        pltpu.VMEM((NG, D), _f32),
        pltpu.VMEM((NG, D), _f32),
    ],
)


def kernel(x, edge_index, batch, W1, b1, W2, b2):
    npad = NEP - NE
    src = jnp.concatenate(
        [edge_index[0].astype(_i32), jnp.zeros((npad,), _i32)])
    dst = jnp.concatenate(
        [edge_index[1].astype(_i32), jnp.full((npad,), NNP - 1, _i32)])
    x_pad = jnp.concatenate([x, jnp.zeros((NNP - NN, D), _f32)])
    batch_pad = jnp.concatenate(
        [batch.astype(_i32), jnp.full((NNP - NN,), NG, _i32)])
    zrow = jnp.zeros((RPT, D), _f32)
    ones = jnp.ones((CH, D), _f32)
    batch_r = batch_pad.reshape(NBLK, BLK, 1)

    degc = _sc_degree(dst, zrow, ones)
    u1, dinvb = _tc1(degc, x_pad, W1)
    aggp1 = _sc_agg(src, dst, u1, zrow)
    u2 = _tc2(aggp1, u1, dinvb, W2, b1.reshape(1, D))
    aggp2 = _sc_agg(src, dst, u2, zrow)
    out = _tc3(aggp2, u2, dinvb, b2.reshape(1, D), batch_r)
    return out
